# baseline (speedup 1.0000x reference)
import functools

import jax
import jax.numpy as jnp
from jax import lax
from jax.experimental import pallas as pl
from jax.experimental.pallas import tpu as pltpu
from jax.experimental.pallas import tpu_sc as plsc

D = 64
H = 4
DH = D // H
NLAYERS = 2


@functools.lru_cache(maxsize=None)
def _build_sc_gather(B, NBR):
    info = plsc.get_sparse_core_info()
    NC, NS = info.num_cores, info.num_subcores
    NW = NC * NS
    total = B * NBR
    pw = total // NW
    CH = 100
    NCH = pw // CH
    assert pw % CH == 0 and total % NW == 0 and B % NW == 0
    sb = B // NW

    mesh = plsc.VectorSubcoreMesh(core_axis_name="c", subcore_axis_name="s")

    @functools.partial(
        pl.kernel,
        mesh=mesh,
        compiler_params=pltpu.CompilerParams(use_tc_tiling_on_sc=False),
        out_type=[
            jax.ShapeDtypeStruct((total, D), jnp.float32),
            jax.ShapeDtypeStruct((total, D), jnp.float32),
            jax.ShapeDtypeStruct((B, D), jnp.float32),
            jax.ShapeDtypeStruct((B, D), jnp.float32),
            jax.ShapeDtypeStruct((B, D), jnp.float32),
            jax.ShapeDtypeStruct((B, 16), jnp.float32),
            jax.ShapeDtypeStruct((B, 16), jnp.float32),
        ],
        scratch_types=[
            pltpu.VMEM((NCH, CH), jnp.int32),
            pltpu.VMEM((pw, D), jnp.float32),
            pltpu.VMEM((sb,), jnp.int32),
            pltpu.VMEM((sb, D), jnp.float32),
            pltpu.VMEM((sb, 16), jnp.float32),
            pltpu.SemaphoreType.DMA,
        ],
    )
    def sc_gather(soc_idx, prod_idx, u_idx, p_idx, c_idx,
                  user_emb, prod_emb, cat_emb, user_b, prod_b,
                  soc_out, prod_out, ue_out, pe_out, ce_out, ub_out, pb_out,
                  idx_v, rows_v, sidx_v, srows_v, b16_v, sem):
        wid = lax.axis_index("s") * NC + lax.axis_index("c")
        for idx3, table, out in ((soc_idx, user_emb, soc_out),
                                 (prod_idx, prod_emb, prod_out)):
            pltpu.sync_copy(idx3.at[wid], idx_v)
            cps = [pltpu.async_copy(table.at[idx_v.at[c]],
                                    rows_v.at[pl.ds(c * CH, CH)], sem)
                   for c in range(NCH)]
            for cp in cps:
                cp.wait()
            pltpu.sync_copy(rows_v, out.at[pl.ds(wid * pw, pw)])
        for idx2, table, out in ((u_idx, user_emb, ue_out),
                                 (p_idx, prod_emb, pe_out),
                                 (c_idx, cat_emb, ce_out)):
            pltpu.sync_copy(idx2.at[wid], sidx_v)
            pltpu.async_copy(table.at[sidx_v], srows_v, sem).wait()
            pltpu.sync_copy(srows_v, out.at[pl.ds(wid * sb, sb)])
        for idx2, table16, out in ((u_idx, user_b, ub_out),
                                   (p_idx, prod_b, pb_out)):
            pltpu.sync_copy(idx2.at[wid], sidx_v)
            cps = []
            for k in range(sb // 16):
                v = sidx_v[pl.ds(k * 16, 16)]
                rows = lax.shift_right_logical(v, 4)
                cps.append(pltpu.async_copy(
                    table16.at[rows], b16_v.at[pl.ds(k * 16, 16)], sem))
            for cp in cps:
                cp.wait()
            pltpu.sync_copy(b16_v, out.at[pl.ds(wid * sb, sb)])

    return sc_gather, NW, NCH, CH, sb


def _tc_body(soc_ref, prod_ref, ue_ref, pe_ref, ce_ref,
             ub16_ref, pb16_ref, uoff_ref, poff_ref, gb_ref,
             sWq_r, sbq_r, sWk_r, sWv_r, sbv_r, sWo_r, sbo_r,
             iWq_r, ibq_r, iWk_r, iWv_r, ibv_r, iWo_r, ibo_r,
             fW_r, fb_r, hW1_r, hb1_r, hW2t_r, hb2_r,
             ratings_ref, help_ref):
    f32 = jnp.float32
    ue = ue_ref[...]
    nbs = (soc_ref[...], prod_ref[...])
    inv_scale = 1.0 / (DH ** 0.5)

    last_parts = None
    fW = fW_r[...]
    fb = fb_r[...]
    for l in range(NLAYERS):
        ctxs = []
        for ci, nb in enumerate(nbs):
            if ci == 0:
                Wq3, bq2, Wk3 = sWq_r[...], sbq_r[...], sWk_r[...]
                Wv3, bv2, Wo3, bo2 = sWv_r[...], sbv_r[...], sWo_r[...], sbo_r[...]
            else:
                Wq3, bq2, Wk3 = iWq_r[...], ibq_r[...], iWk_r[...]
                Wv3, bv2, Wo3, bo2 = iWv_r[...], ibv_r[...], iWo_r[...], ibo_r[...]
            Wq_l, Wk_l, Wv_l, Wo_l = Wq3[l], Wk3[l], Wv3[l], Wo3[l]
            bq_l = bq2[l:l + 1, :]
            att = None
            for h in range(H):
                s0 = h * DH
                Wq_h = Wq_l[:, s0:s0 + DH]
                Wk_h = Wk_l[:, s0:s0 + DH]
                A = lax.dot_general(Wq_h, Wk_h, (((1,), (1,)), ((), ())),
                                    preferred_element_type=f32)
                cvec = lax.dot_general(bq_l[:, s0:s0 + DH], Wk_h,
                                       (((1,), (1,)), ((), ())),
                                       preferred_element_type=f32)
                T = jnp.dot(ue, A, preferred_element_type=f32) + cvec
                s = jnp.sum(nb * T[None, :, :], axis=2) * inv_scale
                m = jnp.max(s, axis=0, keepdims=True)
                e = jnp.exp(s - m)
                w = e / jnp.sum(e, axis=0, keepdims=True)
                ctx = jnp.sum(nb * w[:, :, None], axis=0)
                U = jnp.dot(Wv_l[:, s0:s0 + DH], Wo_l[s0:s0 + DH, :],
                            preferred_element_type=f32)
                contrib = jnp.dot(ctx, U, preferred_element_type=f32)
                att = contrib if att is None else att + contrib
            att = att + (jnp.dot(bv2[l:l + 1, :], Wo_l,
                                 preferred_element_type=f32)
                         + bo2[l:l + 1, :])
            ctxs.append(att)
        f3 = fW[l]
        last_parts = (ue, ctxs[0], ctxs[1])
        ue = jnp.maximum(
            jnp.dot(ue, f3[0:D], preferred_element_type=f32)
            + jnp.dot(ctxs[0], f3[D:2 * D], preferred_element_type=f32)
            + jnp.dot(ctxs[1], f3[2 * D:3 * D], preferred_element_type=f32)
            + fb[l:l + 1, :], 0.0)

    cpe = pe_ref[...] + 0.3 * ce_ref[...]
    inter = jnp.sum(ue * cpe, axis=1)
    g = gb_ref[0, 0]
    lane = lax.broadcasted_iota(jnp.int32, ub16_ref.shape, 1)
    ubv = jnp.sum(jnp.where(lane == uoff_ref[0, 0, :][:, None],
                            ub16_ref[...], 0.0), axis=1)
    pbv = jnp.sum(jnp.where(lane == poff_ref[0, 0, :][:, None],
                            pb16_ref[...], 0.0), axis=1)
    ratings_ref[0, 0, :] = inter + ubv + pbv + g

    hW1 = hW1_r[...]
    u0, c0, c1 = last_parts
    a1 = jnp.maximum(
        jnp.dot(u0, hW1[0:D], preferred_element_type=f32)
        + jnp.dot(c0, hW1[D:2 * D], preferred_element_type=f32)
        + jnp.dot(c1, hW1[2 * D:3 * D], preferred_element_type=f32)
        + hb1_r[...], 0.0)
    z = jnp.sum(a1 * hW2t_r[...], axis=1) + hb2_r[0, 0]
    help_ref[0, 0, :] = jax.nn.sigmoid(z)


def _tc_compute(soc3, prod3, ue0, pe, ce, ub16, pb16, uoff3, poff3, gb2,
                soc_Wq, soc_bq, soc_Wk, soc_Wv, soc_bv, soc_Wo, soc_bo,
                intr_Wq, intr_bq, intr_Wk, intr_Wv, intr_bv, intr_Wo, intr_bo,
                fus_W, fus_b, h_W1, hb1_2, hW2t, hb2_2,
                interpret=False):
    NBR, B, _ = soc3.shape
    NBLK = 8
    BB = B // NBLK
    L3D = fus_W.shape[1]

    def full(shape):
        return pl.BlockSpec(shape, lambda i: tuple(0 for _ in shape))

    in_specs = [
        pl.BlockSpec((NBR, BB, D), lambda i: (0, i, 0)),
        pl.BlockSpec((NBR, BB, D), lambda i: (0, i, 0)),
        pl.BlockSpec((BB, D), lambda i: (i, 0)),
        pl.BlockSpec((BB, D), lambda i: (i, 0)),
        pl.BlockSpec((BB, D), lambda i: (i, 0)),
        pl.BlockSpec((BB, 16), lambda i: (i, 0)),
        pl.BlockSpec((BB, 16), lambda i: (i, 0)),
        pl.BlockSpec((1, 1, BB), lambda i: (i, 0, 0)),
        pl.BlockSpec((1, 1, BB), lambda i: (i, 0, 0)),
        full((1, 1)),
        full((NLAYERS, D, D)), full((NLAYERS, D)), full((NLAYERS, D, D)),
        full((NLAYERS, D, D)), full((NLAYERS, D)), full((NLAYERS, D, D)),
        full((NLAYERS, D)),
        full((NLAYERS, D, D)), full((NLAYERS, D)), full((NLAYERS, D, D)),
        full((NLAYERS, D, D)), full((NLAYERS, D)), full((NLAYERS, D, D)),
        full((NLAYERS, D)),
        full((NLAYERS, L3D, D)), full((NLAYERS, D)),
        full((L3D, D)), full((1, D)), full((1, D)), full((1, 1)),
    ]
    out_specs = [
        pl.BlockSpec((1, 1, BB), lambda i: (i, 0, 0)),
        pl.BlockSpec((1, 1, BB), lambda i: (i, 0, 0)),
    ]
    out_shape = [
        jax.ShapeDtypeStruct((NBLK, 1, BB), jnp.float32),
        jax.ShapeDtypeStruct((NBLK, 1, BB), jnp.float32),
    ]
    r2, h2 = pl.pallas_call(
        _tc_body,
        grid=(NBLK,),
        in_specs=in_specs,
        out_specs=out_specs,
        out_shape=out_shape,
        interpret=interpret,
    )(soc3, prod3, ue0, pe, ce, ub16, pb16, uoff3, poff3, gb2,
      soc_Wq, soc_bq, soc_Wk, soc_Wv, soc_bv, soc_Wo, soc_bo,
      intr_Wq, intr_bq, intr_Wk, intr_Wv, intr_bv, intr_Wo, intr_bo,
      fus_W, fus_b, h_W1, hb1_2, hW2t, hb2_2)
    return r2.reshape(B), h2.reshape(B)


def kernel(user_idx, product_idx, category_idx, social_neighbors,
           product_neighbors, user_emb, prod_emb, cat_emb, user_b, prod_b,
           global_b, soc_Wq, soc_bq, soc_Wk, soc_bk, soc_Wv, soc_bv, soc_Wo,
           soc_bo, intr_Wq, intr_bq, intr_Wk, intr_bk, intr_Wv, intr_bv,
           intr_Wo, intr_bo, fus_W, fus_b, h_W1, h_b1, h_W2, h_b2):
    B = user_idx.shape[0]
    NBR = social_neighbors.shape[1]
    sc_gather, NW, NCH, CH, sb = _build_sc_gather(B, NBR)

    i32 = jnp.int32
    soc_idx3 = social_neighbors.astype(i32).T.reshape(NW, NCH, CH)
    prod_idx3 = product_neighbors.astype(i32).T.reshape(NW, NCH, CH)
    u_idx2 = user_idx.astype(i32).reshape(NW, sb)
    p_idx2 = product_idx.astype(i32).reshape(NW, sb)
    c_idx2 = category_idx.astype(i32).reshape(NW, sb)

    soc_rows, prod_rows, ue0, pe, ce, ub_rows, pb_rows = sc_gather(
        soc_idx3, prod_idx3, u_idx2, p_idx2, c_idx2,
        user_emb, prod_emb, cat_emb,
        user_b.reshape(-1, 16), prod_b.reshape(-1, 16))

    NBLK = 8
    BB = B // NBLK
    uoff3 = jnp.bitwise_and(user_idx.astype(i32), 15).reshape(NBLK, 1, BB)
    poff3 = jnp.bitwise_and(product_idx.astype(i32), 15).reshape(NBLK, 1, BB)
    ratings, helpfulness = _tc_compute(
        soc_rows.reshape(NBR, B, D), prod_rows.reshape(NBR, B, D),
        ue0, pe, ce,
        ub_rows, pb_rows, uoff3, poff3,
        global_b.reshape(1, 1),
        soc_Wq, soc_bq, soc_Wk, soc_Wv, soc_bv, soc_Wo, soc_bo,
        intr_Wq, intr_bq, intr_Wk, intr_Wv, intr_bv, intr_Wo, intr_bo,
        fus_W, fus_b, h_W1, h_b1.reshape(1, D), h_W2.reshape(1, D),
        h_b2.reshape(1, 1))
    return ratings, helpfulness


# speedup vs baseline: 1.0650x; 1.0650x over previous
import functools

import jax
import jax.numpy as jnp
from jax import lax
from jax.experimental import pallas as pl
from jax.experimental.pallas import tpu as pltpu
from jax.experimental.pallas import tpu_sc as plsc

D = 64
H = 4
DH = D // H
NLAYERS = 2


@functools.lru_cache(maxsize=None)
def _build_sc_gather(B, NBR):
    info = plsc.get_sparse_core_info()
    NC, NS = info.num_cores, info.num_subcores
    NW = NC * NS
    total = B * NBR
    pw = total // NW
    CH = 100
    NCH = pw // CH
    assert pw % CH == 0 and total % NW == 0 and B % NW == 0
    sb = B // NW

    mesh = plsc.VectorSubcoreMesh(core_axis_name="c", subcore_axis_name="s")

    @functools.partial(
        pl.kernel,
        mesh=mesh,
        compiler_params=pltpu.CompilerParams(use_tc_tiling_on_sc=False),
        out_type=[
            jax.ShapeDtypeStruct((total, D), jnp.float32),
            jax.ShapeDtypeStruct((total, D), jnp.float32),
            jax.ShapeDtypeStruct((B, D), jnp.float32),
            jax.ShapeDtypeStruct((B, D), jnp.float32),
            jax.ShapeDtypeStruct((B, D), jnp.float32),
            jax.ShapeDtypeStruct((B, 16), jnp.float32),
            jax.ShapeDtypeStruct((B, 16), jnp.float32),
        ],
        scratch_types=[
            pltpu.VMEM((NCH, CH), jnp.int32),
            pltpu.VMEM((pw, D), jnp.float32),
            pltpu.VMEM((sb,), jnp.int32),
            pltpu.VMEM((sb, D), jnp.float32),
            pltpu.VMEM((sb, 16), jnp.float32),
            pltpu.SemaphoreType.DMA,
        ],
    )
    def sc_gather(soc_idx, prod_idx, u_idx, p_idx, c_idx,
                  user_emb, prod_emb, cat_emb, user_b, prod_b,
                  soc_out, prod_out, ue_out, pe_out, ce_out, ub_out, pb_out,
                  idx_v, rows_v, sidx_v, srows_v, b16_v, sem):
        wid = lax.axis_index("s") * NC + lax.axis_index("c")
        for idx3, table, out in ((soc_idx, user_emb, soc_out),
                                 (prod_idx, prod_emb, prod_out)):
            pltpu.sync_copy(idx3.at[wid], idx_v)
            cps = [pltpu.async_copy(table.at[idx_v.at[c]],
                                    rows_v.at[pl.ds(c * CH, CH)], sem)
                   for c in range(NCH)]
            for cp in cps:
                cp.wait()
            pltpu.sync_copy(rows_v, out.at[pl.ds(wid * pw, pw)])
        for idx2, table, out in ((u_idx, user_emb, ue_out),
                                 (p_idx, prod_emb, pe_out),
                                 (c_idx, cat_emb, ce_out)):
            pltpu.sync_copy(idx2.at[wid], sidx_v)
            pltpu.async_copy(table.at[sidx_v], srows_v, sem).wait()
            pltpu.sync_copy(srows_v, out.at[pl.ds(wid * sb, sb)])
        for idx2, table16, out in ((u_idx, user_b, ub_out),
                                   (p_idx, prod_b, pb_out)):
            pltpu.sync_copy(idx2.at[wid], sidx_v)
            cps = []
            for k in range(sb // 16):
                v = sidx_v[pl.ds(k * 16, 16)]
                rows = lax.shift_right_logical(v, 4)
                cps.append(pltpu.async_copy(
                    table16.at[rows], b16_v.at[pl.ds(k * 16, 16)], sem))
            for cp in cps:
                cp.wait()
            pltpu.sync_copy(b16_v, out.at[pl.ds(wid * sb, sb)])

    return sc_gather, NW, NCH, CH, sb


def _tc_body(soc_ref, prod_ref, ue_ref, pe_ref, ce_ref,
             ub16_ref, pb16_ref, uoff_ref, poff_ref, gb_ref,
             sWq_r, sbq_r, sWk_r, sWv_r, sbv_r, sWo_r, sbo_r,
             iWq_r, ibq_r, iWk_r, iWv_r, ibv_r, iWo_r, ibo_r,
             fW_r, fb_r, hW1_r, hb1_r, hW2t_r, hb2_r,
             ratings_ref, help_ref):
    f32 = jnp.float32
    ue = ue_ref[...]
    nbs = (soc_ref[...], prod_ref[...])
    inv_scale = 1.0 / (DH ** 0.5)

    last_parts = None
    fW = fW_r[...]
    fb = fb_r[...]
    for l in range(NLAYERS):
        ctxs = []
        for ci, nb in enumerate(nbs):
            if ci == 0:
                Wq3, bq2, Wk3 = sWq_r[...], sbq_r[...], sWk_r[...]
                Wv3, bv2, Wo3, bo2 = sWv_r[...], sbv_r[...], sWo_r[...], sbo_r[...]
            else:
                Wq3, bq2, Wk3 = iWq_r[...], ibq_r[...], iWk_r[...]
                Wv3, bv2, Wo3, bo2 = iWv_r[...], ibv_r[...], iWo_r[...], ibo_r[...]
            Wq_l, Wk_l, Wv_l, Wo_l = Wq3[l], Wk3[l], Wv3[l], Wo3[l]
            bq_l = bq2[l:l + 1, :]
            att = None
            for h in range(H):
                s0 = h * DH
                Wq_h = Wq_l[:, s0:s0 + DH]
                Wk_h = Wk_l[:, s0:s0 + DH]
                A = lax.dot_general(Wq_h, Wk_h, (((1,), (1,)), ((), ())),
                                    preferred_element_type=f32)
                cvec = lax.dot_general(bq_l[:, s0:s0 + DH], Wk_h,
                                       (((1,), (1,)), ((), ())),
                                       preferred_element_type=f32)
                T = jnp.dot(ue, A, preferred_element_type=f32) + cvec
                s = jnp.sum(nb * T[None, :, :], axis=2) * inv_scale
                m = jnp.max(s, axis=0, keepdims=True)
                e = jnp.exp(s - m)
                w = e / jnp.sum(e, axis=0, keepdims=True)
                ctx = jnp.sum(nb * w[:, :, None], axis=0)
                U = jnp.dot(Wv_l[:, s0:s0 + DH], Wo_l[s0:s0 + DH, :],
                            preferred_element_type=f32)
                contrib = jnp.dot(ctx, U, preferred_element_type=f32)
                att = contrib if att is None else att + contrib
            att = att + (jnp.dot(bv2[l:l + 1, :], Wo_l,
                                 preferred_element_type=f32)
                         + bo2[l:l + 1, :])
            ctxs.append(att)
        f3 = fW[l]
        last_parts = (ue, ctxs[0], ctxs[1])
        ue = jnp.maximum(
            jnp.dot(ue, f3[0:D], preferred_element_type=f32)
            + jnp.dot(ctxs[0], f3[D:2 * D], preferred_element_type=f32)
            + jnp.dot(ctxs[1], f3[2 * D:3 * D], preferred_element_type=f32)
            + fb[l:l + 1, :], 0.0)

    cpe = pe_ref[...] + 0.3 * ce_ref[...]
    inter = jnp.sum(ue * cpe, axis=1)
    g = gb_ref[0, 0]
    lane = lax.broadcasted_iota(jnp.int32, ub16_ref.shape, 1)
    ubv = jnp.sum(jnp.where(lane == uoff_ref[0, 0, :][:, None],
                            ub16_ref[...], 0.0), axis=1)
    pbv = jnp.sum(jnp.where(lane == poff_ref[0, 0, :][:, None],
                            pb16_ref[...], 0.0), axis=1)
    ratings_ref[0, 0, :] = inter + ubv + pbv + g

    hW1 = hW1_r[...]
    u0, c0, c1 = last_parts
    a1 = jnp.maximum(
        jnp.dot(u0, hW1[0:D], preferred_element_type=f32)
        + jnp.dot(c0, hW1[D:2 * D], preferred_element_type=f32)
        + jnp.dot(c1, hW1[2 * D:3 * D], preferred_element_type=f32)
        + hb1_r[...], 0.0)
    z = jnp.sum(a1 * hW2t_r[...], axis=1) + hb2_r[0, 0]
    help_ref[0, 0, :] = jax.nn.sigmoid(z)


def _tc_compute(soc3, prod3, ue0, pe, ce, ub16, pb16, uoff3, poff3, gb2,
                soc_Wq, soc_bq, soc_Wk, soc_Wv, soc_bv, soc_Wo, soc_bo,
                intr_Wq, intr_bq, intr_Wk, intr_Wv, intr_bv, intr_Wo, intr_bo,
                fus_W, fus_b, h_W1, hb1_2, hW2t, hb2_2,
                interpret=False):
    NBR, B, _ = soc3.shape
    NBLK = 4
    BB = B // NBLK
    L3D = fus_W.shape[1]

    def full(shape):
        return pl.BlockSpec(shape, lambda i: tuple(0 for _ in shape))

    in_specs = [
        pl.BlockSpec((NBR, BB, D), lambda i: (0, i, 0)),
        pl.BlockSpec((NBR, BB, D), lambda i: (0, i, 0)),
        pl.BlockSpec((BB, D), lambda i: (i, 0)),
        pl.BlockSpec((BB, D), lambda i: (i, 0)),
        pl.BlockSpec((BB, D), lambda i: (i, 0)),
        pl.BlockSpec((BB, 16), lambda i: (i, 0)),
        pl.BlockSpec((BB, 16), lambda i: (i, 0)),
        pl.BlockSpec((1, 1, BB), lambda i: (i, 0, 0)),
        pl.BlockSpec((1, 1, BB), lambda i: (i, 0, 0)),
        full((1, 1)),
        full((NLAYERS, D, D)), full((NLAYERS, D)), full((NLAYERS, D, D)),
        full((NLAYERS, D, D)), full((NLAYERS, D)), full((NLAYERS, D, D)),
        full((NLAYERS, D)),
        full((NLAYERS, D, D)), full((NLAYERS, D)), full((NLAYERS, D, D)),
        full((NLAYERS, D, D)), full((NLAYERS, D)), full((NLAYERS, D, D)),
        full((NLAYERS, D)),
        full((NLAYERS, L3D, D)), full((NLAYERS, D)),
        full((L3D, D)), full((1, D)), full((1, D)), full((1, 1)),
    ]
    out_specs = [
        pl.BlockSpec((1, 1, BB), lambda i: (i, 0, 0)),
        pl.BlockSpec((1, 1, BB), lambda i: (i, 0, 0)),
    ]
    out_shape = [
        jax.ShapeDtypeStruct((NBLK, 1, BB), jnp.float32),
        jax.ShapeDtypeStruct((NBLK, 1, BB), jnp.float32),
    ]
    r2, h2 = pl.pallas_call(
        _tc_body,
        grid=(NBLK,),
        in_specs=in_specs,
        out_specs=out_specs,
        out_shape=out_shape,
        interpret=interpret,
    )(soc3, prod3, ue0, pe, ce, ub16, pb16, uoff3, poff3, gb2,
      soc_Wq, soc_bq, soc_Wk, soc_Wv, soc_bv, soc_Wo, soc_bo,
      intr_Wq, intr_bq, intr_Wk, intr_Wv, intr_bv, intr_Wo, intr_bo,
      fus_W, fus_b, h_W1, hb1_2, hW2t, hb2_2)
    return r2.reshape(B), h2.reshape(B)


def kernel(user_idx, product_idx, category_idx, social_neighbors,
           product_neighbors, user_emb, prod_emb, cat_emb, user_b, prod_b,
           global_b, soc_Wq, soc_bq, soc_Wk, soc_bk, soc_Wv, soc_bv, soc_Wo,
           soc_bo, intr_Wq, intr_bq, intr_Wk, intr_bk, intr_Wv, intr_bv,
           intr_Wo, intr_bo, fus_W, fus_b, h_W1, h_b1, h_W2, h_b2):
    B = user_idx.shape[0]
    NBR = social_neighbors.shape[1]
    sc_gather, NW, NCH, CH, sb = _build_sc_gather(B, NBR)

    i32 = jnp.int32
    soc_idx3 = social_neighbors.astype(i32).T.reshape(NW, NCH, CH)
    prod_idx3 = product_neighbors.astype(i32).T.reshape(NW, NCH, CH)
    u_idx2 = user_idx.astype(i32).reshape(NW, sb)
    p_idx2 = product_idx.astype(i32).reshape(NW, sb)
    c_idx2 = category_idx.astype(i32).reshape(NW, sb)

    soc_rows, prod_rows, ue0, pe, ce, ub_rows, pb_rows = sc_gather(
        soc_idx3, prod_idx3, u_idx2, p_idx2, c_idx2,
        user_emb, prod_emb, cat_emb,
        user_b.reshape(-1, 16), prod_b.reshape(-1, 16))

    NBLK = 4
    BB = B // NBLK
    uoff3 = jnp.bitwise_and(user_idx.astype(i32), 15).reshape(NBLK, 1, BB)
    poff3 = jnp.bitwise_and(product_idx.astype(i32), 15).reshape(NBLK, 1, BB)
    ratings, helpfulness = _tc_compute(
        soc_rows.reshape(NBR, B, D), prod_rows.reshape(NBR, B, D),
        ue0, pe, ce,
        ub_rows, pb_rows, uoff3, poff3,
        global_b.reshape(1, 1),
        soc_Wq, soc_bq, soc_Wk, soc_Wv, soc_bv, soc_Wo, soc_bo,
        intr_Wq, intr_bq, intr_Wk, intr_Wv, intr_bv, intr_Wo, intr_bo,
        fus_W, fus_b, h_W1, h_b1.reshape(1, D), h_W2.reshape(1, D),
        h_b2.reshape(1, 1))
    return ratings, helpfulness
